# baseline (speedup 1.0000x reference)
import jax
import jax.numpy as jnp
from jax import lax
from jax.experimental import pallas as pl
from jax.experimental.pallas import tpu as pltpu
from jax.experimental.pallas import tpu_sc as plsc

_NUM_USERS = 500000
_LANES = 16


def _make_body(batch, emb, nw, half):
    bpw = batch // nw
    ngrp = bpw // _LANES

    def body(id_hbm, eua_hbm, eub_hbm, eia_hbm, eib_hbm, out_hbm, ids_v,
             tbuf, obuf, gsem, osem):
        nc = lax.axis_size("c")
        wid = lax.axis_index("s") * nc + lax.axis_index("c")
        base = wid * bpw

        pltpu.sync_copy(id_hbm.at[pl.ds(base, bpw)], ids_v)

        def fire(g):
            p = g % 2
            idv = ids_v[pl.ds(g * _LANES, _LANES)]
            m = idv < _NUM_USERS
            eff = jnp.where(m, idv, idv - _NUM_USERS)
            effh = jnp.where(eff < half, eff, eff - half)
            for u in range(_LANES):
                s = idv[u]
                t = effh[u] >> 3
                t8 = pl.multiple_of(t << 3, 8)
                dst = tbuf.at[p, u]

                @pl.when(s < half)
                def _():
                    pltpu.async_copy(eua_hbm.at[t], dst, gsem)

                @pl.when(jnp.logical_and(s >= half, s < _NUM_USERS))
                def _():
                    pltpu.async_copy(eub_hbm.at[pl.ds(t8, 8), :], dst, gsem)

                @pl.when(jnp.logical_and(s >= _NUM_USERS,
                                         s < _NUM_USERS + half))
                def _():
                    pltpu.async_copy(eia_hbm.at[t], dst, gsem)

                @pl.when(s >= _NUM_USERS + half)
                def _():
                    pltpu.async_copy(eib_hbm.at[pl.ds(t8, 8), :], dst, gsem)

        def drain_select(g):
            p = g % 2
            for u in range(_LANES):
                pltpu.make_async_copy(eua_hbm.at[0],
                                      tbuf.at[p, u], gsem).wait()
            idv = ids_v[pl.ds(g * _LANES, _LANES)]
            eff = jnp.where(idv < _NUM_USERS, idv, idv - _NUM_USERS)
            sub = eff & 7
            for u in range(_LANES):
                sb = sub[u]
                r = g * _LANES + u
                for cc in range(emb // _LANES):
                    obuf[r, pl.ds(cc * _LANES, _LANES)] = (
                        tbuf[p, u, sb, pl.ds(cc * _LANES, _LANES)])

        fire(0)

        def pipe(g, carry):
            fire(g + 1)
            drain_select(g)
            return carry

        lax.fori_loop(0, ngrp - 1, pipe, 0)
        drain_select(ngrp - 1)
        pltpu.sync_copy(obuf, out_hbm.at[pl.ds(base, bpw)])

    return body, bpw


def kernel(id, e_user, e_item):
    batch = id.shape[0]
    emb = e_user.shape[1]
    half = e_user.shape[0] // 2
    info = plsc.get_sparse_core_info()
    nw = info.num_cores * info.num_subcores
    eua = e_user[:half].reshape(half // 8, 8, emb)
    eub = e_user[half:]
    eia = e_item[:half].reshape(half // 8, 8, emb)
    eib = e_item[half:]
    body, bpw = _make_body(batch, emb, nw, half)
    mesh = plsc.VectorSubcoreMesh(core_axis_name="c", subcore_axis_name="s")
    f = pl.kernel(
        body,
        out_type=jax.ShapeDtypeStruct((batch, emb), jnp.float32),
        mesh=mesh,
        compiler_params=pltpu.CompilerParams(use_tc_tiling_on_sc=True),
        scratch_types=[
            pltpu.VMEM((bpw,), jnp.int32),
            pltpu.VMEM((2, _LANES, 8, emb), jnp.float32),
            pltpu.VMEM((bpw, emb), jnp.float32),
            pltpu.SemaphoreType.DMA,
            pltpu.SemaphoreType.DMA,
        ],
    )
    return f(id, eua, eub, eia, eib)


# speedup vs baseline: 2.0055x; 2.0055x over previous
import jax
import jax.numpy as jnp
from jax import lax
from jax.experimental import pallas as pl
from jax.experimental.pallas import tpu as pltpu
from jax.experimental.pallas import tpu_sc as plsc

_NUM_USERS = 500000
_LANES = 16


def _make_body(batch, emb, nw):
    bpw = batch // nw
    ngrp = bpw // _LANES

    def body(id_hbm, eu_hbm, ei_hbm, out_hbm, ids_v, tbuf, obuf, gsem, osem):
        nc = lax.axis_size("c")
        wid = lax.axis_index("s") * nc + lax.axis_index("c")
        base = wid * bpw

        pltpu.sync_copy(id_hbm.at[pl.ds(base, bpw)], ids_v)

        def fire(g):
            p = g % 2
            idv = ids_v[pl.ds(g * _LANES, _LANES)]
            eff = jnp.where(idv < _NUM_USERS, idv, idv - _NUM_USERS)
            for u in range(_LANES):
                s = idv[u]
                t = eff[u] >> 3

                @pl.when(s < _NUM_USERS)
                def _():
                    pltpu.async_copy(eu_hbm.at[t], tbuf.at[p, u], gsem)

                @pl.when(s >= _NUM_USERS)
                def _():
                    pltpu.async_copy(ei_hbm.at[t], tbuf.at[p, u], gsem)

        def drain_select(g):
            p = g % 2
            for u in range(_LANES):
                pltpu.make_async_copy(eu_hbm.at[0],
                                      tbuf.at[p, u], gsem).wait()
            idv = ids_v[pl.ds(g * _LANES, _LANES)]
            eff = jnp.where(idv < _NUM_USERS, idv, idv - _NUM_USERS)
            sub = eff & 7
            for u in range(_LANES):
                sb = sub[u]
                r = g * _LANES + u
                for cc in range(emb // _LANES):
                    obuf[r, pl.ds(cc * _LANES, _LANES)] = (
                        tbuf[p, u, sb, pl.ds(cc * _LANES, _LANES)])

        fire(0)

        def pipe(g, carry):
            fire(g + 1)
            drain_select(g)
            return carry

        lax.fori_loop(0, ngrp - 1, pipe, 0)
        drain_select(ngrp - 1)
        pltpu.sync_copy(obuf, out_hbm.at[pl.ds(base, bpw)])

    return body, bpw


def kernel(id, e_user, e_item):
    batch = id.shape[0]
    emb = e_user.shape[1]
    info = plsc.get_sparse_core_info()
    nw = info.num_cores * info.num_subcores
    eu3 = e_user.reshape(e_user.shape[0] // 8, 8, emb)
    ei3 = e_item.reshape(e_item.shape[0] // 8, 8, emb)
    body, bpw = _make_body(batch, emb, nw)
    mesh = plsc.VectorSubcoreMesh(core_axis_name="c", subcore_axis_name="s")
    f = pl.kernel(
        body,
        out_type=jax.ShapeDtypeStruct((batch, emb), jnp.float32),
        mesh=mesh,
        compiler_params=pltpu.CompilerParams(use_tc_tiling_on_sc=True),
        scratch_types=[
            pltpu.VMEM((bpw,), jnp.int32),
            pltpu.VMEM((2, _LANES, 8, emb), jnp.float32),
            pltpu.VMEM((bpw, emb), jnp.float32),
            pltpu.SemaphoreType.DMA,
            pltpu.SemaphoreType.DMA,
        ],
    )
    return f(id, eu3, ei3)


# speedup vs baseline: 2.0465x; 1.0205x over previous
import jax
import jax.numpy as jnp
from jax import lax
from jax.experimental import pallas as pl
from jax.experimental.pallas import tpu as pltpu
from jax.experimental.pallas import tpu_sc as plsc

_NUM_USERS = 500000
_LANES = 16


def _make_body(batch, emb, nw):
    bpw = batch // nw
    ngrp = bpw // _LANES

    def body(id_hbm, eu_hbm, ei_hbm, out_hbm, ids_v, tbuf, obuf, gsem, osem):
        nc = lax.axis_size("c")
        wid = lax.axis_index("s") * nc + lax.axis_index("c")
        base = wid * bpw

        pltpu.sync_copy(id_hbm.at[pl.ds(base, bpw)], ids_v)

        def fire(g):
            p = g % 3
            idv = ids_v[pl.ds(g * _LANES, _LANES)]
            eff = jnp.where(idv < _NUM_USERS, idv, idv - _NUM_USERS)
            for u in range(_LANES):
                s = idv[u]
                t = eff[u] >> 3

                @pl.when(s < _NUM_USERS)
                def _():
                    pltpu.async_copy(eu_hbm.at[t], tbuf.at[p, u], gsem)

                @pl.when(s >= _NUM_USERS)
                def _():
                    pltpu.async_copy(ei_hbm.at[t], tbuf.at[p, u], gsem)

        def drain_select(g):
            p = g % 3
            for u in range(_LANES):
                pltpu.make_async_copy(eu_hbm.at[0],
                                      tbuf.at[p, u], gsem).wait()
            idv = ids_v[pl.ds(g * _LANES, _LANES)]
            eff = jnp.where(idv < _NUM_USERS, idv, idv - _NUM_USERS)
            sub = eff & 7
            for u in range(_LANES):
                sb = sub[u]
                r = g * _LANES + u
                for cc in range(emb // _LANES):
                    obuf[r, pl.ds(cc * _LANES, _LANES)] = (
                        tbuf[p, u, sb, pl.ds(cc * _LANES, _LANES)])

        fire(0)
        fire(1)

        def pipe(g, carry):
            fire(g + 2)
            drain_select(g)
            return carry

        lax.fori_loop(0, ngrp - 2, pipe, 0)
        drain_select(ngrp - 2)
        drain_select(ngrp - 1)
        pltpu.sync_copy(obuf, out_hbm.at[pl.ds(base, bpw)])

    return body, bpw


def kernel(id, e_user, e_item):
    batch = id.shape[0]
    emb = e_user.shape[1]
    info = plsc.get_sparse_core_info()
    nw = info.num_cores * info.num_subcores
    eu3 = e_user.reshape(e_user.shape[0] // 8, 8, emb)
    ei3 = e_item.reshape(e_item.shape[0] // 8, 8, emb)
    body, bpw = _make_body(batch, emb, nw)
    mesh = plsc.VectorSubcoreMesh(core_axis_name="c", subcore_axis_name="s")
    f = pl.kernel(
        body,
        out_type=jax.ShapeDtypeStruct((batch, emb), jnp.float32),
        mesh=mesh,
        compiler_params=pltpu.CompilerParams(use_tc_tiling_on_sc=True),
        scratch_types=[
            pltpu.VMEM((bpw,), jnp.int32),
            pltpu.VMEM((3, _LANES, 8, emb), jnp.float32),
            pltpu.VMEM((bpw, emb), jnp.float32),
            pltpu.SemaphoreType.DMA,
            pltpu.SemaphoreType.DMA,
        ],
    )
    return f(id, eu3, ei3)
